# Initial kernel scaffold; baseline (speedup 1.0000x reference)
import jax
import jax.numpy as jnp
from jax.experimental import pallas as pl


def kernel(x, position_embeddings):
    raise NotImplementedError("write your pallas kernel here")


# speedup vs baseline: 3.5421x; 3.5421x over previous
import functools

import jax
import jax.numpy as jnp
from jax import lax
from jax.experimental import pallas as pl
from jax.experimental.pallas import tpu as pltpu
from jax.experimental.pallas import tpu_sc as plsc

_B, _S, _D = 4, 8192, 1024
_NC, _NS = 2, 16
_NW = _NC * _NS
_RPW = _S // _NW
_CH = 32
_NCHUNK = _RPW // _CH


def _make_replicate():
    mesh = plsc.VectorSubcoreMesh(core_axis_name="c", subcore_axis_name="s")

    @functools.partial(
        pl.kernel,
        mesh=mesh,
        out_type=jax.ShapeDtypeStruct((_B, _S, _D), jnp.float32),
        scratch_types=[
            pltpu.VMEM((_CH, _D), jnp.float32),
            pltpu.VMEM((_CH, _D), jnp.float32),
            pltpu.SemaphoreType.DMA,
            pltpu.SemaphoreType.DMA,
        ],
    )
    def body(table_hbm, out_hbm, buf0, buf1, rsem, wsem):
        wid = lax.axis_index("s") * _NC + lax.axis_index("c")
        base = wid * _RPW
        bufs = (buf0, buf1)

        def read(g, buf):
            return pltpu.async_copy(
                table_hbm.at[pl.ds(base + g * _CH, _CH)], buf, rsem)

        def write(g, buf):
            return [
                pltpu.async_copy(
                    buf, out_hbm.at[b, pl.ds(base + g * _CH, _CH)], wsem)
                for b in range(_B)
            ]

        pending = {}
        rd = read(0, buf0)
        for g in range(_NCHUNK):
            nxt = None
            if g + 1 < _NCHUNK:
                if g - 1 in pending:
                    for w in pending.pop(g - 1):
                        w.wait()
                nxt = read(g + 1, bufs[(g + 1) % 2])
            rd.wait()
            pending[g] = write(g, bufs[g % 2])
            rd = nxt
        for g in sorted(pending):
            for w in pending.pop(g):
                w.wait()

    return body


_replicate = _make_replicate()


def kernel(x, position_embeddings):
    del x
    return _replicate(position_embeddings)


# speedup vs baseline: 5.0363x; 1.4218x over previous
import functools

import jax
import jax.numpy as jnp
from jax import lax
from jax.experimental import pallas as pl
from jax.experimental.pallas import tpu as pltpu
from jax.experimental.pallas import tpu_sc as plsc

_B, _S, _D = 4, 8192, 1024
_NC, _NS = 2, 16
_NW = _NC * _NS
_RPW = _S // _NW
_CH = 32
_NCHUNK = _RPW // _CH


def _make_replicate():
    mesh = plsc.VectorSubcoreMesh(core_axis_name="c", subcore_axis_name="s")

    @functools.partial(
        pl.kernel,
        mesh=mesh,
        out_type=jax.ShapeDtypeStruct((_B, _S, _D), jnp.float32),
        scratch_types=[
            pltpu.VMEM((_CH, _D), jnp.float32),
            pltpu.VMEM((_CH, _D), jnp.float32),
            pltpu.SemaphoreType.DMA,
            pltpu.SemaphoreType.DMA,
        ],
    )
    def body(table_hbm, out_hbm, buf0, buf1, rsem, wsem):
        wid = lax.axis_index("s") * _NC + lax.axis_index("c")
        base = wid * _RPW
        bufs = (buf0, buf1)

        def read(g, buf):
            return pltpu.async_copy(
                table_hbm.at[pl.ds(base + g * _CH, _CH)], buf, rsem)

        def write(g, buf):
            return [
                pltpu.async_copy(
                    buf, out_hbm.at[b, pl.ds(base + g * _CH, _CH)], wsem)
                for b in range(_B)
            ]

        pending = {}
        rd = read(0, buf0)
        for g in range(_NCHUNK):
            nxt = None
            if g + 1 < _NCHUNK:
                if g - 1 in pending:
                    for w in pending.pop(g - 1):
                        w.wait()
                nxt = read(g + 1, bufs[(g + 1) % 2])
            rd.wait()
            pending[g] = write(g, bufs[g % 2])
            rd = nxt
        for g in sorted(pending):
            for w in pending.pop(g):
                w.wait()

    return body


_replicate = _make_replicate()

_TC_CH = 512


def _tc_body(t_ref, o_ref):
    o_ref[...] = jnp.broadcast_to(t_ref[...][None], (_B, _TC_CH, _D))


def _tc_replicate(table):
    return pl.pallas_call(
        _tc_body,
        grid=(_S // _TC_CH,),
        in_specs=[pl.BlockSpec((_TC_CH, _D), lambda i: (i, 0))],
        out_specs=pl.BlockSpec((_B, _TC_CH, _D), lambda i: (0, i, 0)),
        out_shape=jax.ShapeDtypeStruct((_B, _S, _D), jnp.float32),
    )(table)


def kernel(x, position_embeddings):
    del x
    return _tc_replicate(position_embeddings)
